# baseline (speedup 1.0000x reference)
import functools

import jax
import jax.numpy as jnp
from jax import lax
from jax.experimental import pallas as pl
from jax.experimental.pallas import tpu as pltpu
from jax.experimental.pallas import tpu_sc as plsc

T = 4096
D = 1024
FF = 512
E = 8
K = 2
NA = T * K
BMG = 256
NBLK = NA // BMG + E
NSLOT = NBLK * BMG
NW = 16
APW = NA // NW
SPW = NSLOT // NW


def _router_kernel(x_ref, w_ref, idx_ref, val_ref):
    logits = jnp.dot(x_ref[:], w_ref[:], preferred_element_type=jnp.float32)
    m = jnp.max(logits, axis=-1, keepdims=True)
    ex = jnp.exp(logits - m)
    probs = ex / jnp.sum(ex, axis=-1, keepdims=True)
    ids = jax.lax.broadcasted_iota(jnp.int32, probs.shape, 1)
    big = jnp.int32(E)
    v1 = jnp.max(probs, axis=-1, keepdims=True)
    i1 = jnp.min(jnp.where(probs == v1, ids, big), axis=-1, keepdims=True)
    m1 = ids == i1
    p2 = jnp.where(m1, -jnp.inf, probs)
    v2 = jnp.max(p2, axis=-1, keepdims=True)
    i2 = jnp.min(jnp.where(p2 == v2, ids, big), axis=-1, keepdims=True)
    s = v1 + v2
    col = jax.lax.broadcasted_iota(jnp.int32, (probs.shape[0], K), 1)
    idx_ref[:] = jnp.where(col == 0, i1, i2)
    val_ref[:] = jnp.where(col == 0, v1 / s, v2 / s)


def _router(x, router_w):
    bm = 2048
    return pl.pallas_call(
        _router_kernel,
        grid=(T // bm,),
        in_specs=[
            pl.BlockSpec((bm, D), lambda i: (i, 0)),
            pl.BlockSpec((D, E), lambda i: (0, 0)),
        ],
        out_specs=[
            pl.BlockSpec((bm, K), lambda i: (i, 0)),
            pl.BlockSpec((bm, K), lambda i: (i, 0)),
        ],
        out_shape=[
            jax.ShapeDtypeStruct((T, K), jnp.int32),
            jax.ShapeDtypeStruct((T, K), jnp.float32),
        ],
    )(x, router_w)


def _dispatch_body(eflat_hbm, tok_hbm, slot_hbm, blk_hbm, counts_hbm,
                   ev_buf, pos_buf, tok_buf, zb, cnt_stage,
                   call_vm, blk_vm, sem):
    wid = lax.axis_index("s")
    base_a = wid * APW
    zeros = jnp.zeros((16,), jnp.int32)
    lane = lax.iota(jnp.int32, 16)

    for i in range(SPW // 16):
        zb[pl.ds(16 * i, 16)] = zeros
    pltpu.sync_copy(zb, tok_hbm.at[pl.ds(wid * SPW, SPW)])

    pltpu.sync_copy(eflat_hbm.at[pl.ds(base_a, APW)], ev_buf)
    cnt = zeros
    for c in range(APW // 16):
        ev = ev_buf[pl.ds(16 * c, 16)]
        for e in range(E):
            pc = plsc.all_reduce_population_count(ev == e)
            cnt = cnt + jnp.where(lane == e, pc, zeros)
    cnt_stage[...] = cnt
    pltpu.sync_copy(cnt_stage, counts_hbm.at[wid])
    plsc.subcore_barrier()

    pltpu.sync_copy(counts_hbm, call_vm)
    tot = zeros
    for w2 in range(NW):
        tot = tot + call_vm[w2]
    padded = ((tot + (BMG - 1)) // BMG) * BMG
    incl = jnp.cumsum(padded)
    base = incl - padded
    start = base
    for w2 in range(NW):
        row = call_vm[w2]
        start = start + jnp.where(jnp.int32(w2) < wid, row, zeros)

    @pl.when(wid == 0)
    def _blk():
        for v in range(NBLK // 16 + 1):
            bs = (lane + 16 * v) * BMG
            acc = jnp.full((16,), -1, jnp.int32)
            for e in range(E):
                acc = acc + jnp.where(bs >= base[e], 1, 0).astype(jnp.int32)
            blk_vm[pl.ds(16 * v, 16)] = jnp.minimum(acc, E - 1)
        pltpu.sync_copy(blk_vm, blk_hbm)

    for c in range(APW // 16):
        ev = ev_buf[pl.ds(16 * c, 16)]
        rank = zeros
        bse = zeros
        inc = zeros
        for e in range(E):
            m = ev == e
            cs = jnp.cumsum(m.astype(jnp.int32))
            rank = jnp.where(m, cs - 1, rank)
            bse = jnp.where(m, start[e], bse)
            pc = plsc.all_reduce_population_count(m)
            inc = inc + jnp.where(lane == e, pc, zeros)
        pos = bse + rank
        r, col = c // 8, 16 * (c % 8)
        pos_buf[r, pl.ds(col, 16)] = pos
        av = base_a + 16 * c + lane
        tok_buf[r, pl.ds(col, 16)] = av // K
        start = start + inc

    for c in range(APW // 16):
        r, col = c // 8, 16 * (c % 8)
        pos16 = pos_buf[r, pl.ds(col, 16)]
        pltpu.async_copy(
            tok_buf.at[r, pl.ds(col, 16)], tok_hbm.at[pos16], sem).wait()
    pltpu.sync_copy(pos_buf, slot_hbm.at[pl.ds(wid * (APW // 128), APW // 128)])


def _dispatch(eflat):
    mesh = plsc.VectorSubcoreMesh(
        core_axis_name="c", subcore_axis_name="s", num_cores=1,
        num_subcores=16)
    fn = pl.kernel(
        _dispatch_body,
        mesh=mesh,
        compiler_params=pltpu.CompilerParams(needs_layout_passes=False),
        out_type=[
            jax.ShapeDtypeStruct((NSLOT,), jnp.int32),
            jax.ShapeDtypeStruct((NA // 128, 128), jnp.int32),
            jax.ShapeDtypeStruct((48,), jnp.int32),
            jax.ShapeDtypeStruct((NW, 16), jnp.int32),
        ],
        scratch_types=[
            pltpu.VMEM((APW,), jnp.int32),
            pltpu.VMEM((APW // 128, 128), jnp.int32),
            pltpu.VMEM((APW // 128, 128), jnp.int32),
            pltpu.VMEM((SPW,), jnp.int32),
            pltpu.VMEM((16,), jnp.int32),
            pltpu.VMEM((NW, 16), jnp.int32),
            pltpu.VMEM((48,), jnp.int32),
            pltpu.SemaphoreType.DMA,
        ],
    )
    tok_sorted, slot2d, blk48, _ = fn(eflat)
    return tok_sorted, slot2d, blk48


def _gather_body(xb_hbm, tok_hbm, xg_hbm, idx_vm, rows_vm, sem):
    wid = lax.axis_index("s") * 2 + lax.axis_index("c")
    spw = NSLOT // 32
    base = wid * spw
    pltpu.sync_copy(tok_hbm.at[pl.ds(base, spw)], idx_vm)
    for j in range(spw // 64):
        for jj in range(4):
            idx16 = idx_vm[pl.ds(64 * j + 16 * jj, 16)]
            pltpu.async_copy(
                xb_hbm.at[idx16], rows_vm.at[pl.ds(16 * jj, 16)], sem).wait()
        pltpu.sync_copy(rows_vm, xg_hbm.at[pl.ds(base + 64 * j, 64)])


def _gather(xb_i32, tok_sorted):
    mesh = plsc.VectorSubcoreMesh(core_axis_name="c", subcore_axis_name="s",
                                  num_cores=2, num_subcores=16)
    spw = NSLOT // 32
    fn = pl.kernel(
        _gather_body,
        mesh=mesh,
        compiler_params=pltpu.CompilerParams(needs_layout_passes=False),
        out_type=jax.ShapeDtypeStruct((NSLOT, D // 2), jnp.int32),
        scratch_types=[
            pltpu.VMEM((spw,), jnp.int32),
            pltpu.VMEM((64, D // 2), jnp.int32),
            pltpu.SemaphoreType.DMA,
        ],
    )
    return fn(xb_i32, tok_sorted)


def _gemm_body(eid_ref, xg_ref, gu_ref, dp_ref, yg_ref):
    del eid_ref
    h = jnp.dot(xg_ref[:], gu_ref[0], preferred_element_type=jnp.float32)
    g = h[:, :FF]
    u = h[:, FF:]
    act = (g * jax.nn.sigmoid(g) * u).astype(jnp.bfloat16)
    yg_ref[:] = jnp.dot(act, dp_ref[0], preferred_element_type=jnp.float32)


def _gemm(blk_eid, xg, gu, dp):
    grid_spec = pltpu.PrefetchScalarGridSpec(
        num_scalar_prefetch=1,
        grid=(NBLK,),
        in_specs=[
            pl.BlockSpec((BMG, D), lambda b, eid: (b, 0)),
            pl.BlockSpec((1, D, 2 * FF), lambda b, eid: (eid[b], 0, 0)),
            pl.BlockSpec((1, FF, D), lambda b, eid: (eid[b], 0, 0)),
        ],
        out_specs=pl.BlockSpec((BMG, D), lambda b, eid: (b, 0)),
    )
    return pl.pallas_call(
        _gemm_body,
        grid_spec=grid_spec,
        out_shape=jax.ShapeDtypeStruct((NSLOT, D), jnp.float32),
        compiler_params=pltpu.CompilerParams(
            dimension_semantics=("arbitrary",),
        ),
    )(blk_eid, xg, gu, dp)


def _combine_body(slot_hbm, w_hbm, yg_hbm, out_hbm,
                  idx_vm, w_vm, gbuf, obuf, sem):
    wid = lax.axis_index("s") * 2 + lax.axis_index("c")
    for c in range(8):
        row = wid * 8 + c
        pltpu.sync_copy(slot_hbm.at[row], idx_vm)
        pltpu.sync_copy(w_hbm.at[row], w_vm)
        idx16a = idx_vm[pl.ds(0, 16)]
        idx16b = idx_vm[pl.ds(16, 16)]
        pltpu.async_copy(yg_hbm.at[idx16a], gbuf.at[pl.ds(0, 16)], sem).wait()
        pltpu.async_copy(yg_hbm.at[idx16b], gbuf.at[pl.ds(16, 16)], sem).wait()
        wv0 = w_vm[pl.ds(0, 16)]
        wv1 = w_vm[pl.ds(16, 16)]
        w0s = [(wv0 if 2 * i < 16 else wv1)[(2 * i) % 16] for i in range(16)]
        w1s = [(wv0 if 2 * i + 1 < 16 else wv1)[(2 * i + 1) % 16]
               for i in range(16)]

        def jbody(j, carry):
            sl = pl.ds(16 * j, 16)
            for i in range(16):
                obuf[i, sl] = (w0s[i] * gbuf[2 * i, sl]
                               + w1s[i] * gbuf[2 * i + 1, sl])
            return carry

        lax.fori_loop(0, D // 16, jbody, 0)
        pltpu.sync_copy(obuf, out_hbm.at[pl.ds(wid * 128 + 16 * c, 16)])


def _combine(slot2d, w2d, yg):
    mesh = plsc.VectorSubcoreMesh(core_axis_name="c", subcore_axis_name="s",
                                  num_cores=2, num_subcores=16)
    fn = pl.kernel(
        _combine_body,
        mesh=mesh,
        compiler_params=pltpu.CompilerParams(needs_layout_passes=False),
        out_type=jax.ShapeDtypeStruct((T, D), jnp.float32),
        scratch_types=[
            pltpu.VMEM((32,), jnp.int32),
            pltpu.VMEM((32,), jnp.float32),
            pltpu.VMEM((32, D), jnp.float32),
            pltpu.VMEM((16, D), jnp.float32),
            pltpu.SemaphoreType.DMA,
        ],
    )
    return fn(slot2d, w2d, yg)


def kernel(x, router_w, gate_up_proj, down_proj):
    top_idx, top_val = _router(x, router_w)
    eflat = top_idx.reshape(NA)
    tok_sorted, slot2d, blk48 = _dispatch(eflat)

    xb = x.astype(jnp.bfloat16)
    xb_i32 = lax.bitcast_convert_type(
        xb.reshape(T, D // 2, 2), jnp.int32)
    xg_i32 = _gather(xb_i32, tok_sorted)
    xg = lax.bitcast_convert_type(xg_i32, jnp.bfloat16).reshape(NSLOT, D)

    gu = gate_up_proj.astype(jnp.bfloat16)
    dp = down_proj.astype(jnp.bfloat16)
    yg = _gemm(blk48[:NBLK], xg, gu, dp)

    slot_r = slot2d.reshape(NA // 32, 32)
    w_r = top_val.reshape(NA // 32, 32)
    return _combine(slot_r, w_r, yg)
